# baseline (speedup 1.0000x reference)
import functools

import jax
import jax.numpy as jnp
import numpy as np
from jax import lax
from jax.experimental import pallas as pl
from jax.experimental.pallas import tpu as pltpu
from jax.experimental.pallas import tpu_sc as plsc

VOCAB = 10000
EMBED_DIM = 256
BATCH = 4096
HIST = 50

NUM_CORES = 2
NUM_SUBCORES = 16
NUM_WORKERS = NUM_CORES * NUM_SUBCORES
BPW = BATCH // NUM_WORKERS
RPC = 2
PAIRS = HIST // 2
PAIR_UNROLL = 5
NBUF = 6
NTAIL = (BATCH // NUM_WORKERS // RPC) % NBUF
NSPLIT = 1
ROWS = BATCH // NSPLIT
BPW_S = ROWS // NUM_WORKERS
CPW = BPW_S // RPC
PKD = EMBED_DIM // 2
DBLK = PKD // 16

_HI_MASK = np.int32(-65536)


def _pool_body(idx_hbm, table_hbm, out_hbm, idx_v, buf0, buf1, buf2, buf3,
               buf4, buf5, out_v, sem0, sem1, sem2, sem3, sem4, sem5):
    c = lax.axis_index("c")
    s = lax.axis_index("s")
    wid = c * NUM_SUBCORES + s
    base = wid * BPW_S

    bufs = (buf0, buf1, buf2, buf3, buf4, buf5)
    sems = (sem0, sem1, sem2, sem3, sem4, sem5)

    pltpu.sync_copy(idx_hbm.at[wid], idx_v)

    for t in range(NBUF - 1):
        pltpu.async_copy(table_hbm.at[idx_v.at[t]], bufs[t], sems[t])

    def reduce_chunk(buf, t):
        for rr in range(RPC):
            r0 = rr * HIST

            def jbody(g, carry):
                accs = list(carry)
                for u in range(PAIR_UNROLL):
                    j = PAIR_UNROLL * g + u
                    for blk in range(DBLK):
                        a = buf[r0 + 2 * j, pl.ds(blk * 16, 16)]
                        b = buf[r0 + 2 * j + 1, pl.ds(blk * 16, 16)]
                        sbf = (plsc.bitcast(a, jnp.bfloat16)
                               + plsc.bitcast(b, jnp.bfloat16))
                        v = plsc.bitcast(sbf, jnp.int32)
                        lo = plsc.bitcast(lax.shift_left(v, 16), jnp.float32)
                        hi = plsc.bitcast(lax.bitwise_and(v, _HI_MASK),
                                          jnp.float32)
                        accs[2 * blk] = accs[2 * blk] + lo
                        accs[2 * blk + 1] = accs[2 * blk + 1] + hi
                return tuple(accs)

            zero = jnp.zeros((16,), jnp.float32)
            accs = lax.fori_loop(0, PAIRS // PAIR_UNROLL, jbody,
                                 tuple(zero for _ in range(2 * DBLK)))
            row = RPC * t + rr
            for blk in range(DBLK):
                out_v[row, pl.ds(blk * 32, 16)] = accs[2 * blk]
                out_v[row, pl.ds(blk * 32 + 16, 16)] = accs[2 * blk + 1]

    def body(k, carry):
        for sub in range(NBUF):
            t = NBUF * k + sub
            pltpu.make_async_copy(
                table_hbm.at[idx_v.at[t]], bufs[sub], sems[sub]).wait()

            @pl.when(t + NBUF - 1 < CPW)
            def _():
                pltpu.async_copy(table_hbm.at[idx_v.at[t + NBUF - 1]],
                                 bufs[(sub + NBUF - 1) % NBUF],
                                 sems[(sub + NBUF - 1) % NBUF])

            reduce_chunk(bufs[sub], t)
        return carry

    lax.fori_loop(0, CPW // NBUF, body, 0)

    for sub in range(NTAIL):
        t = CPW - NTAIL + sub
        pltpu.make_async_copy(
            table_hbm.at[idx_v.at[t]], bufs[t % NBUF], sems[t % NBUF]).wait()
        reduce_chunk(bufs[t % NBUF], t)

    pltpu.sync_copy(out_v, out_hbm.at[pl.ds(base, BPW_S)])


@functools.cache
def _pool():
    return pl.kernel(
        _pool_body,
        out_type=jax.ShapeDtypeStruct((ROWS, EMBED_DIM), jnp.float32),
        mesh=plsc.VectorSubcoreMesh(
            core_axis_name="c", subcore_axis_name="s",
            num_cores=NUM_CORES, num_subcores=NUM_SUBCORES,
        ),
        scratch_types=[
            pltpu.VMEM((CPW, RPC * HIST), jnp.int32),
            pltpu.VMEM((RPC * HIST, PKD), jnp.int32),
            pltpu.VMEM((RPC * HIST, PKD), jnp.int32),
            pltpu.VMEM((RPC * HIST, PKD), jnp.int32),
            pltpu.VMEM((RPC * HIST, PKD), jnp.int32),
            pltpu.VMEM((RPC * HIST, PKD), jnp.int32),
            pltpu.VMEM((RPC * HIST, PKD), jnp.int32),
            pltpu.VMEM((BPW_S, EMBED_DIM), jnp.float32),
            pltpu.SemaphoreType.DMA,
            pltpu.SemaphoreType.DMA,
            pltpu.SemaphoreType.DMA,
            pltpu.SemaphoreType.DMA,
            pltpu.SemaphoreType.DMA,
            pltpu.SemaphoreType.DMA,
        ],
        compiler_params=pltpu.CompilerParams(needs_layout_passes=False),
    )


def _pack_body(t_ref, o_ref):
    t = t_ref[...]
    lo = lax.bitcast_convert_type(
        t[:, :PKD].astype(jnp.bfloat16), jnp.uint16).astype(jnp.uint32)
    hi = lax.bitcast_convert_type(
        t[:, PKD:].astype(jnp.bfloat16), jnp.uint16).astype(jnp.uint32)
    o_ref[...] = lax.bitcast_convert_type(
        lax.bitwise_or(lax.shift_left(hi, jnp.uint32(16)), lo), jnp.int32)


def _pack(table):
    blk = 1000
    grid = VOCAB // blk
    return pl.pallas_call(
        _pack_body,
        grid=(grid,),
        in_specs=[pl.BlockSpec((blk, EMBED_DIM), lambda i: (i, 0))],
        out_specs=pl.BlockSpec((blk, PKD), lambda i: (i, 0)),
        out_shape=jax.ShapeDtypeStruct((VOCAB, PKD), jnp.int32),
    )(table)


def _head_body(p_ref, w_ref, b_ref, o_ref):
    p = p_ref[...]
    h = lax.dot_general(
        p, w_ref[...], (((1,), (1,)), ((), ())),
        preferred_element_type=jnp.float32,
    )
    h = h * (1.0 / HIST) + b_ref[...]
    norm = jnp.sqrt(jnp.sum(h * h, axis=1, keepdims=True))
    o_ref[...] = h / jnp.maximum(norm, 1e-12)


def _head(pooled_sum, Wp, b2d):
    blk = 512
    grid = ROWS // blk
    return pl.pallas_call(
        _head_body,
        grid=(grid,),
        in_specs=[
            pl.BlockSpec((blk, EMBED_DIM), lambda i: (i, 0)),
            pl.BlockSpec((EMBED_DIM, EMBED_DIM), lambda i: (0, 0)),
            pl.BlockSpec((1, EMBED_DIM), lambda i: (0, 0)),
        ],
        out_specs=pl.BlockSpec((blk, EMBED_DIM), lambda i: (i, 0)),
        out_shape=jax.ShapeDtypeStruct((ROWS, EMBED_DIM), jnp.float32),
    )(pooled_sum, Wp, b2d)


@jax.jit
def kernel(x, table, W, b):
    idx = x.astype(jnp.int32).reshape(NUM_WORKERS, CPW, RPC * HIST)
    table_pk = _pack(table)
    Wp = W.reshape(EMBED_DIM, 2, 8, 16).transpose(0, 2, 1, 3).reshape(
        EMBED_DIM, EMBED_DIM)
    pooled_sum = _pool()(idx, table_pk)
    return _head(pooled_sum, Wp, b.reshape(1, EMBED_DIM))


# speedup vs baseline: 1.0593x; 1.0593x over previous
import functools

import jax
import jax.numpy as jnp
import numpy as np
from jax import lax
from jax.experimental import pallas as pl
from jax.experimental.pallas import tpu as pltpu
from jax.experimental.pallas import tpu_sc as plsc

VOCAB = 10000
EMBED_DIM = 256
BATCH = 4096
HIST = 50

NUM_CORES = 2
NUM_SUBCORES = 16
NUM_WORKERS = NUM_CORES * NUM_SUBCORES
BPW = BATCH // NUM_WORKERS
RPC = 2
PAIRS = HIST // 2
PAIR_UNROLL = 5
NBUF = 4
NTAIL = (BATCH // NUM_WORKERS // RPC) % NBUF
NSPLIT = 1
ROWS = BATCH // NSPLIT
BPW_S = ROWS // NUM_WORKERS
CPW = BPW_S // RPC
PKD = EMBED_DIM // 2
DBLK = PKD // 16

_HI_MASK = np.int32(-65536)


def _pool_body(idx_hbm, table_hbm, out_hbm, idx_v, buf0, buf1, buf2, buf3,
               out_v, sem0, sem1, sem2, sem3):
    c = lax.axis_index("c")
    s = lax.axis_index("s")
    wid = c * NUM_SUBCORES + s
    base = wid * BPW_S

    bufs = (buf0, buf1, buf2, buf3)
    sems = (sem0, sem1, sem2, sem3)

    pltpu.sync_copy(idx_hbm.at[wid], idx_v)

    for t in range(NBUF - 1):
        pltpu.async_copy(table_hbm.at[idx_v.at[t]], bufs[t], sems[t])

    def reduce_chunk(buf, t):
        for rr in range(RPC):
            r0 = rr * HIST

            def jbody(g, carry):
                accs = list(carry)
                for u in range(PAIR_UNROLL):
                    j = PAIR_UNROLL * g + u
                    for blk in range(DBLK):
                        a = buf[r0 + 2 * j, pl.ds(blk * 16, 16)]
                        b = buf[r0 + 2 * j + 1, pl.ds(blk * 16, 16)]
                        sbf = (plsc.bitcast(a, jnp.bfloat16)
                               + plsc.bitcast(b, jnp.bfloat16))
                        v = plsc.bitcast(sbf, jnp.int32)
                        lo = plsc.bitcast(lax.shift_left(v, 16), jnp.float32)
                        hi = plsc.bitcast(lax.bitwise_and(v, _HI_MASK),
                                          jnp.float32)
                        accs[2 * blk] = accs[2 * blk] + lo
                        accs[2 * blk + 1] = accs[2 * blk + 1] + hi
                return tuple(accs)

            zero = jnp.zeros((16,), jnp.float32)
            accs = lax.fori_loop(0, PAIRS // PAIR_UNROLL, jbody,
                                 tuple(zero for _ in range(2 * DBLK)))
            row = RPC * t + rr
            for blk in range(DBLK):
                out_v[row, pl.ds(blk * 32, 16)] = accs[2 * blk]
                out_v[row, pl.ds(blk * 32 + 16, 16)] = accs[2 * blk + 1]

    def body(k, carry):
        for sub in range(NBUF):
            t = NBUF * k + sub
            pltpu.make_async_copy(
                table_hbm.at[idx_v.at[t]], bufs[sub], sems[sub]).wait()

            @pl.when(t + NBUF - 1 < CPW)
            def _():
                pltpu.async_copy(table_hbm.at[idx_v.at[t + NBUF - 1]],
                                 bufs[(sub + NBUF - 1) % NBUF],
                                 sems[(sub + NBUF - 1) % NBUF])

            reduce_chunk(bufs[sub], t)
        return carry

    lax.fori_loop(0, CPW // NBUF, body, 0)

    for sub in range(NTAIL):
        t = CPW - NTAIL + sub
        pltpu.make_async_copy(
            table_hbm.at[idx_v.at[t]], bufs[t % NBUF], sems[t % NBUF]).wait()
        reduce_chunk(bufs[t % NBUF], t)

    pltpu.sync_copy(out_v, out_hbm.at[pl.ds(base, BPW_S)])


@functools.cache
def _pool():
    return pl.kernel(
        _pool_body,
        out_type=jax.ShapeDtypeStruct((ROWS, EMBED_DIM), jnp.float32),
        mesh=plsc.VectorSubcoreMesh(
            core_axis_name="c", subcore_axis_name="s",
            num_cores=NUM_CORES, num_subcores=NUM_SUBCORES,
        ),
        scratch_types=[
            pltpu.VMEM((CPW, RPC * HIST), jnp.int32),
            pltpu.VMEM((RPC * HIST, PKD), jnp.int32),
            pltpu.VMEM((RPC * HIST, PKD), jnp.int32),
            pltpu.VMEM((RPC * HIST, PKD), jnp.int32),
            pltpu.VMEM((RPC * HIST, PKD), jnp.int32),
            pltpu.VMEM((BPW_S, EMBED_DIM), jnp.float32),
            pltpu.SemaphoreType.DMA,
            pltpu.SemaphoreType.DMA,
            pltpu.SemaphoreType.DMA,
            pltpu.SemaphoreType.DMA,
        ],
        compiler_params=pltpu.CompilerParams(needs_layout_passes=False),
    )


def _pack_body(t_ref, o_ref):
    t = t_ref[...]
    lo = lax.bitcast_convert_type(
        t[:, :PKD].astype(jnp.bfloat16), jnp.uint16).astype(jnp.uint32)
    hi = lax.bitcast_convert_type(
        t[:, PKD:].astype(jnp.bfloat16), jnp.uint16).astype(jnp.uint32)
    o_ref[...] = lax.bitcast_convert_type(
        lax.bitwise_or(lax.shift_left(hi, jnp.uint32(16)), lo), jnp.int32)


def _pack(table):
    blk = 1000
    grid = VOCAB // blk
    return pl.pallas_call(
        _pack_body,
        grid=(grid,),
        in_specs=[pl.BlockSpec((blk, EMBED_DIM), lambda i: (i, 0))],
        out_specs=pl.BlockSpec((blk, PKD), lambda i: (i, 0)),
        out_shape=jax.ShapeDtypeStruct((VOCAB, PKD), jnp.int32),
    )(table)


def _head_body(p_ref, w_ref, b_ref, o_ref):
    p = p_ref[...]
    h = lax.dot_general(
        p, w_ref[...], (((1,), (1,)), ((), ())),
        preferred_element_type=jnp.float32,
    )
    h = h * (1.0 / HIST) + b_ref[...]
    norm = jnp.sqrt(jnp.sum(h * h, axis=1, keepdims=True))
    o_ref[...] = h / jnp.maximum(norm, 1e-12)


def _head(pooled_sum, Wp, b2d):
    blk = 512
    grid = ROWS // blk
    return pl.pallas_call(
        _head_body,
        grid=(grid,),
        in_specs=[
            pl.BlockSpec((blk, EMBED_DIM), lambda i: (i, 0)),
            pl.BlockSpec((EMBED_DIM, EMBED_DIM), lambda i: (0, 0)),
            pl.BlockSpec((1, EMBED_DIM), lambda i: (0, 0)),
        ],
        out_specs=pl.BlockSpec((blk, EMBED_DIM), lambda i: (i, 0)),
        out_shape=jax.ShapeDtypeStruct((ROWS, EMBED_DIM), jnp.float32),
    )(pooled_sum, Wp, b2d)


@jax.jit
def kernel(x, table, W, b):
    idx = x.astype(jnp.int32).reshape(NUM_WORKERS, CPW, RPC * HIST)
    table_pk = _pack(table)
    Wp = W.reshape(EMBED_DIM, 2, 8, 16).transpose(0, 2, 1, 3).reshape(
        EMBED_DIM, EMBED_DIM)
    pooled_sum = _pool()(idx, table_pk)
    return _head(pooled_sum, Wp, b.reshape(1, EMBED_DIM))


# speedup vs baseline: 1.0641x; 1.0046x over previous
import functools

import jax
import jax.numpy as jnp
import numpy as np
from jax import lax
from jax.experimental import pallas as pl
from jax.experimental.pallas import tpu as pltpu
from jax.experimental.pallas import tpu_sc as plsc

VOCAB = 10000
EMBED_DIM = 256
BATCH = 4096
HIST = 50

NUM_CORES = 2
NUM_SUBCORES = 16
NUM_WORKERS = NUM_CORES * NUM_SUBCORES
RPC = 2
PAIRS = HIST // 2
PAIR_UNROLL = 5
NBUF = 4
ROWS = BATCH
BPW_S = ROWS // NUM_WORKERS
CPW = BPW_S // RPC
NTAIL = CPW % NBUF
PKD = EMBED_DIM // 2
DBLK = PKD // 16

_HI_MASK = np.int32(-65536)


def _pool_body(idx_hbm, table_hbm, out_hbm, idx_v, buf0, buf1, buf2, buf3,
               out_v, sem0, sem1, sem2, sem3):
    c = lax.axis_index("c")
    s = lax.axis_index("s")
    wid = c * NUM_SUBCORES + s
    base = wid * BPW_S

    bufs = (buf0, buf1, buf2, buf3)
    sems = (sem0, sem1, sem2, sem3)

    pltpu.sync_copy(idx_hbm.at[wid], idx_v)

    for t in range(NBUF - 1):
        pltpu.async_copy(table_hbm.at[idx_v.at[t]], bufs[t], sems[t])

    def reduce_chunk(buf, t):
        for rr in range(RPC):
            r0 = rr * HIST

            def jbody(g, carry):
                accs = list(carry)
                for u in range(PAIR_UNROLL):
                    j = PAIR_UNROLL * g + u
                    for blk in range(DBLK):
                        a = buf[r0 + 2 * j, pl.ds(blk * 16, 16)]
                        b = buf[r0 + 2 * j + 1, pl.ds(blk * 16, 16)]
                        sbf = (plsc.bitcast(a, jnp.bfloat16)
                               + plsc.bitcast(b, jnp.bfloat16))
                        v = plsc.bitcast(sbf, jnp.int32)
                        lo = plsc.bitcast(lax.shift_left(v, 16), jnp.float32)
                        hi = plsc.bitcast(lax.bitwise_and(v, _HI_MASK),
                                          jnp.float32)
                        accs[2 * blk] = accs[2 * blk] + lo
                        accs[2 * blk + 1] = accs[2 * blk + 1] + hi
                return tuple(accs)

            zero = jnp.zeros((16,), jnp.float32)
            accs = lax.fori_loop(0, PAIRS // PAIR_UNROLL, jbody,
                                 tuple(zero for _ in range(2 * DBLK)))
            row = RPC * t + rr
            for blk in range(DBLK):
                out_v[row, pl.ds(blk * 32, 16)] = accs[2 * blk]
                out_v[row, pl.ds(blk * 32 + 16, 16)] = accs[2 * blk + 1]

    def body(k, carry):
        for sub in range(NBUF):
            t = NBUF * k + sub
            pltpu.make_async_copy(
                table_hbm.at[idx_v.at[t]], bufs[sub], sems[sub]).wait()

            @pl.when(t + NBUF - 1 < CPW)
            def _():
                pltpu.async_copy(table_hbm.at[idx_v.at[t + NBUF - 1]],
                                 bufs[(sub + NBUF - 1) % NBUF],
                                 sems[(sub + NBUF - 1) % NBUF])

            reduce_chunk(bufs[sub], t)
        return carry

    lax.fori_loop(0, CPW // NBUF, body, 0)

    for sub in range(NTAIL):
        t = CPW - NTAIL + sub
        pltpu.make_async_copy(
            table_hbm.at[idx_v.at[t]], bufs[t % NBUF], sems[t % NBUF]).wait()
        reduce_chunk(bufs[t % NBUF], t)

    pltpu.sync_copy(out_v, out_hbm.at[pl.ds(base, BPW_S)])


@functools.cache
def _pool():
    return pl.kernel(
        _pool_body,
        out_type=jax.ShapeDtypeStruct((ROWS, EMBED_DIM), jnp.float32),
        mesh=plsc.VectorSubcoreMesh(
            core_axis_name="c", subcore_axis_name="s",
            num_cores=NUM_CORES, num_subcores=NUM_SUBCORES,
        ),
        scratch_types=[
            pltpu.VMEM((CPW, RPC * HIST), jnp.int32),
            pltpu.VMEM((RPC * HIST, PKD), jnp.int32),
            pltpu.VMEM((RPC * HIST, PKD), jnp.int32),
            pltpu.VMEM((RPC * HIST, PKD), jnp.int32),
            pltpu.VMEM((RPC * HIST, PKD), jnp.int32),
            pltpu.VMEM((BPW_S, EMBED_DIM), jnp.float32),
            pltpu.SemaphoreType.DMA,
            pltpu.SemaphoreType.DMA,
            pltpu.SemaphoreType.DMA,
            pltpu.SemaphoreType.DMA,
        ],
        compiler_params=pltpu.CompilerParams(needs_layout_passes=False),
    )


def _pack_body(t_ref, o_ref):
    t = t_ref[...]
    lo = lax.bitcast_convert_type(
        t[:, :PKD].astype(jnp.bfloat16), jnp.uint16).astype(jnp.uint32)
    hi = lax.bitcast_convert_type(
        t[:, PKD:].astype(jnp.bfloat16), jnp.uint16).astype(jnp.uint32)
    o_ref[...] = lax.bitcast_convert_type(
        lax.bitwise_or(lax.shift_left(hi, jnp.uint32(16)), lo), jnp.int32)


def _pack(table):
    blk = 1000
    grid = VOCAB // blk
    return pl.pallas_call(
        _pack_body,
        grid=(grid,),
        in_specs=[pl.BlockSpec((blk, EMBED_DIM), lambda i: (i, 0))],
        out_specs=pl.BlockSpec((blk, PKD), lambda i: (i, 0)),
        out_shape=jax.ShapeDtypeStruct((VOCAB, PKD), jnp.int32),
    )(table)


def _head_body(p_ref, w_ref, b_ref, o_ref):
    p = p_ref[...]
    h = lax.dot_general(
        p, w_ref[...], (((1,), (1,)), ((), ())),
        preferred_element_type=jnp.float32,
    )
    h = h * (1.0 / HIST) + b_ref[...]
    norm = jnp.sqrt(jnp.sum(h * h, axis=1, keepdims=True))
    o_ref[...] = h / jnp.maximum(norm, 1e-12)


def _head(pooled_sum, Wp, b2d):
    blk = 512
    grid = ROWS // blk
    return pl.pallas_call(
        _head_body,
        grid=(grid,),
        in_specs=[
            pl.BlockSpec((blk, EMBED_DIM), lambda i: (i, 0)),
            pl.BlockSpec((EMBED_DIM, EMBED_DIM), lambda i: (0, 0)),
            pl.BlockSpec((1, EMBED_DIM), lambda i: (0, 0)),
        ],
        out_specs=pl.BlockSpec((blk, EMBED_DIM), lambda i: (i, 0)),
        out_shape=jax.ShapeDtypeStruct((ROWS, EMBED_DIM), jnp.float32),
    )(pooled_sum, Wp, b2d)


@jax.jit
def kernel(x, table, W, b):
    idx = x.astype(jnp.int32).reshape(NUM_WORKERS, CPW, RPC * HIST)
    table_pk = _pack(table)
    Wp = W.reshape(EMBED_DIM, 2, 8, 16).transpose(0, 2, 1, 3).reshape(
        EMBED_DIM, EMBED_DIM)
    pooled_sum = _pool()(idx, table_pk)
    return _head(pooled_sum, Wp, b.reshape(1, EMBED_DIM))
